# Initial kernel scaffold; baseline (speedup 1.0000x reference)
import jax
import jax.numpy as jnp
from jax import lax
from jax.experimental import pallas as pl
from jax.experimental.pallas import tpu as pltpu
from jax.experimental.pallas import tpu_sc as plsc

N = 10000
D = 256
H = 128
E = 160000
NS = 16
B = 80
EPT = E // NS
ITERS = EPT // B
RPT = N // NS


def _sc_body(xcat, e0, e1, zacc, zdeg,
             agg0, agg1, deg0, deg1,
             acc_sh, deg_sh, idx_v, srca_v, rows_v, ones_v, sem):
    c = lax.axis_index("c")
    s = lax.axis_index("s")
    r0 = s * RPT
    ebase = s * EPT

    def _fill(i, _):
        ones_v[i] = jnp.ones((16,), jnp.float32)
        return ()
    lax.fori_loop(0, B, _fill, ())

    for r, (e_hbm, a_hbm, d_hbm) in enumerate(
            ((e0, agg0, deg0), (e1, agg1, deg1))):
        pltpu.sync_copy(zacc.at[pl.ds(r0, RPT)], acc_sh.at[pl.ds(r0, RPT)])

        @pl.when(c == r)
        def _():
            pltpu.sync_copy(zdeg.at[pl.ds(r0, RPT)], deg_sh.at[pl.ds(r0, RPT)])

        plsc.subcore_barrier()

        def _edge_iter(it, _, e_hbm=e_hbm, r=r):
            off = ebase + it * B
            pltpu.sync_copy(e_hbm.at[:, pl.ds(off, B)], idx_v)
            for j in range(B // 16):
                srca_v[pl.ds(j * 16, 16)] = idx_v[0, pl.ds(j * 16, 16)] * 2 + c
            pltpu.async_copy(xcat.at[srca_v], rows_v, sem).wait()
            pltpu.sync_copy(rows_v, acc_sh.at[idx_v.at[1]], add=True)

            @pl.when(c == r)
            def _():
                pltpu.sync_copy(ones_v, deg_sh.at[idx_v.at[1]], add=True)

            return ()

        lax.fori_loop(0, ITERS, _edge_iter, ())
        plsc.subcore_barrier()

        pltpu.sync_copy(acc_sh.at[pl.ds(r0, RPT)],
                        a_hbm.at[pl.ds(r0, RPT), pl.ds(c * H, H)])

        @pl.when(c == r)
        def _():
            pltpu.sync_copy(deg_sh.at[pl.ds(r0, RPT)], d_hbm.at[pl.ds(r0, RPT)])

        plsc.subcore_barrier()


def _sc_aggregate(xcat, e0, e1):
    zacc = jnp.zeros((N, H), jnp.float32)
    zdeg = jnp.zeros((N, 16), jnp.float32)
    mesh = plsc.VectorSubcoreMesh(core_axis_name="c", subcore_axis_name="s")
    f = pl.kernel(
        _sc_body,
        out_type=(
            jax.ShapeDtypeStruct((N, D), jnp.float32),
            jax.ShapeDtypeStruct((N, D), jnp.float32),
            jax.ShapeDtypeStruct((N, 16), jnp.float32),
            jax.ShapeDtypeStruct((N, 16), jnp.float32),
        ),
        mesh=mesh,
        scratch_types=[
            pltpu.VMEM_SHARED((N, H), jnp.float32),
            pltpu.VMEM_SHARED((N, 16), jnp.float32),
            pltpu.VMEM((2, B), jnp.int32),
            pltpu.VMEM((B,), jnp.int32),
            pltpu.VMEM((B, H), jnp.float32),
            pltpu.VMEM((B, 16), jnp.float32),
            pltpu.SemaphoreType.DMA,
        ],
    )
    return f(xcat, e0, e1, zacc, zdeg)


def _tc_body(a0, a1, d0, d1, w0, w1, bb0, bb1, o):
    n0 = 1.0 / jnp.maximum(d0[:, 0:1], 1.0)
    n1 = 1.0 / jnp.maximum(d1[:, 0:1], 1.0)
    acc = jnp.dot(a0[...] * n0, w0[...], preferred_element_type=jnp.float32)
    acc += jnp.dot(a1[...] * n1, w1[...], preferred_element_type=jnp.float32)
    o[...] = acc + bb0[...] + bb1[...]


def _tc_epilogue(agg0, agg1, deg0, deg1, W0, b0, W1, b1):
    R = 1000
    grid = (N // R,)
    return pl.pallas_call(
        _tc_body,
        grid=grid,
        in_specs=[
            pl.BlockSpec((R, D), lambda i: (i, 0)),
            pl.BlockSpec((R, D), lambda i: (i, 0)),
            pl.BlockSpec((R, 16), lambda i: (i, 0)),
            pl.BlockSpec((R, 16), lambda i: (i, 0)),
            pl.BlockSpec((D, D), lambda i: (0, 0)),
            pl.BlockSpec((D, D), lambda i: (0, 0)),
            pl.BlockSpec((1, D), lambda i: (0, 0)),
            pl.BlockSpec((1, D), lambda i: (0, 0)),
        ],
        out_specs=pl.BlockSpec((R, D), lambda i: (i, 0)),
        out_shape=jax.ShapeDtypeStruct((N, D), jnp.float32),
    )(agg0, agg1, deg0, deg1, W0, W1,
      b0.reshape(1, D), b1.reshape(1, D))


@jax.jit
def kernel(x, edge_index_rel0, edge_index_rel1, W0, b0, W1, b1):
    xcat = x.reshape(2 * N, H)
    agg0, agg1, deg0, deg1 = _sc_aggregate(xcat, edge_index_rel0,
                                           edge_index_rel1)
    return _tc_epilogue(agg0, agg1, deg0, deg1, W0, b0, W1, b1)


# speedup vs baseline: 2.7958x; 2.7958x over previous
import jax
import jax.numpy as jnp
from jax import lax
from jax.experimental import pallas as pl
from jax.experimental.pallas import tpu as pltpu
from jax.experimental.pallas import tpu_sc as plsc

N = 10000
N_PAD = 10240
D = 256
H = 128
E = 160000
NS = 16
B = 80
EPT = E // NS
ITERS = EPT // B
RPT = N_PAD // NS


def _sc_body(xcat, eboth, zacc, ones,
             agg0, agg1, dg,
             acc_sh, dst_v, srca_v, rows_v, ones_v, sem):
    c = lax.axis_index("c")
    s = lax.axis_index("s")
    r0 = s * RPT
    ebase = s * EPT

    pltpu.sync_copy(ones, ones_v)

    for r, a_hbm in ((0, agg0), (1, agg1)):
        pltpu.sync_copy(zacc.at[pl.ds(r0, RPT)], acc_sh.at[pl.ds(r0, RPT)])
        plsc.subcore_barrier()

        def _edge_iter(it, _, r=r):
            off = ebase + it * B
            pltpu.sync_copy(eboth.at[pl.ds(r * 2 * E + E + off, B)], dst_v)
            pltpu.sync_copy(eboth.at[pl.ds(r * 2 * E + off, B)], srca_v)
            for j in range(B // 16):
                sl = pl.ds(j * 16, 16)
                srca_v[sl] = srca_v[sl] * 2 + c

            pltpu.async_copy(xcat.at[srca_v], rows_v, sem).wait()
            pltpu.sync_copy(rows_v, acc_sh.at[dst_v], add=True)
            return ()

        lax.fori_loop(0, ITERS, _edge_iter, ())
        plsc.subcore_barrier()

        pltpu.sync_copy(acc_sh.at[pl.ds(r0, RPT)],
                        a_hbm.at[c, pl.ds(r0, RPT)])
        plsc.subcore_barrier()

    pltpu.sync_copy(zacc.at[pl.ds(r0, RPT)], acc_sh.at[pl.ds(r0, RPT)])
    plsc.subcore_barrier()

    def _deg_iter(it, _):
        off = c * 2 * E + E + ebase + it * B
        pltpu.sync_copy(eboth.at[pl.ds(off, B)], dst_v)
        pltpu.sync_copy(ones_v, acc_sh.at[dst_v], add=True)
        return ()

    lax.fori_loop(0, ITERS, _deg_iter, ())
    plsc.subcore_barrier()
    pltpu.sync_copy(acc_sh.at[pl.ds(r0, RPT)], dg.at[c, pl.ds(r0, RPT)])


def _sc_aggregate(xcat, eboth):
    zacc = jnp.zeros((N_PAD, H), jnp.float32)
    ones = jnp.ones((B, H), jnp.float32)
    mesh = plsc.VectorSubcoreMesh(core_axis_name="c", subcore_axis_name="s")
    f = pl.kernel(
        _sc_body,
        out_type=(
            jax.ShapeDtypeStruct((2, N_PAD, H), jnp.float32),
            jax.ShapeDtypeStruct((2, N_PAD, H), jnp.float32),
            jax.ShapeDtypeStruct((2, N_PAD, H), jnp.float32),
        ),
        mesh=mesh,
        scratch_types=[
            pltpu.VMEM_SHARED((N_PAD, H), jnp.float32),
            pltpu.VMEM((B,), jnp.int32),
            pltpu.VMEM((B,), jnp.int32),
            pltpu.VMEM((B, H), jnp.float32),
            pltpu.VMEM((B, H), jnp.float32),
            pltpu.SemaphoreType.DMA,
        ],
    )
    return f(xcat, eboth, zacc, ones)


def _tc_body(a0, a1, d0, d1, w0, w1, bb0, bb1, o):
    n0 = 1.0 / jnp.maximum(d0[...], 1.0)
    n1 = 1.0 / jnp.maximum(d1[...], 1.0)
    acc = jnp.dot(a0[...] * n0, w0[...], preferred_element_type=jnp.float32)
    acc += jnp.dot(a1[...] * n1, w1[...], preferred_element_type=jnp.float32)
    o[...] = acc + bb0[...] + bb1[...]


def _tc_epilogue(agg0, agg1, deg0, deg1, W0, b0, W1, b1):
    R = 1000
    grid = (N // R,)
    return pl.pallas_call(
        _tc_body,
        grid=grid,
        in_specs=[
            pl.BlockSpec((R, D), lambda i: (i, 0)),
            pl.BlockSpec((R, D), lambda i: (i, 0)),
            pl.BlockSpec((R, 1), lambda i: (i, 0)),
            pl.BlockSpec((R, 1), lambda i: (i, 0)),
            pl.BlockSpec((D, D), lambda i: (0, 0)),
            pl.BlockSpec((D, D), lambda i: (0, 0)),
            pl.BlockSpec((1, D), lambda i: (0, 0)),
            pl.BlockSpec((1, D), lambda i: (0, 0)),
        ],
        out_specs=pl.BlockSpec((R, D), lambda i: (i, 0)),
        out_shape=jax.ShapeDtypeStruct((N, D), jnp.float32),
    )(agg0, agg1, deg0, deg1, W0, W1,
      b0.reshape(1, D), b1.reshape(1, D))


@jax.jit
def kernel(x, edge_index_rel0, edge_index_rel1, W0, b0, W1, b1):
    xcat = x.reshape(2 * N, H)
    eboth = jnp.concatenate([edge_index_rel0.reshape(2 * E),
                             edge_index_rel1.reshape(2 * E)])
    agg0, agg1, dg = _sc_aggregate(xcat, eboth)
    a0 = jnp.concatenate([agg0[0, :N], agg0[1, :N]], axis=1)
    a1 = jnp.concatenate([agg1[0, :N], agg1[1, :N]], axis=1)
    deg0 = dg[0, :N, 0:1]
    deg1 = dg[1, :N, 0:1]
    return _tc_epilogue(a0, a1, deg0, deg1, W0, b0, W1, b1)


# speedup vs baseline: 4.0381x; 1.4444x over previous
import jax
import jax.numpy as jnp
from jax import lax
from jax.experimental import pallas as pl
from jax.experimental.pallas import tpu as pltpu
from jax.experimental.pallas import tpu_sc as plsc

N = 10000
N_PAD = 10240
D = 256
H = 128
E = 160000
NS = 16
B = 80
EPT = E // NS
ITERS = EPT // B
RPT = N_PAD // NS


def _sc_body(xcat, eboth, zacc, ones,
             agg0, agg1, dg,
             acc_sh, dst0_v, srca0_v, rows0_v, dst1_v, srca1_v, rows1_v,
             ones_v, sem0, sem1):
    c = lax.axis_index("c")
    s = lax.axis_index("s")
    r0 = s * RPT
    ebase = s * EPT

    pltpu.sync_copy(ones, ones_v)

    def _load_idx(r, it, dst_v, srca_v):
        off = ebase + it * B
        pltpu.sync_copy(eboth.at[pl.ds(r * 2 * E + E + off, B)], dst_v)
        pltpu.sync_copy(eboth.at[pl.ds(r * 2 * E + off, B)], srca_v)
        for j in range(B // 16):
            sl = pl.ds(j * 16, 16)
            srca_v[sl] = srca_v[sl] * 2 + c

    for r, a_hbm in ((0, agg0), (1, agg1)):
        pltpu.sync_copy(zacc.at[pl.ds(r0, RPT)], acc_sh.at[pl.ds(r0, RPT)])
        plsc.subcore_barrier()

        _load_idx(r, 0, dst0_v, srca0_v)
        pltpu.async_copy(xcat.at[srca0_v], rows0_v, sem0)

        def _edge_pair(p, _, r=r):
            _load_idx(r, 2 * p + 1, dst1_v, srca1_v)
            pltpu.async_copy(xcat.at[srca1_v], rows1_v, sem1)
            pltpu.make_async_copy(xcat.at[srca0_v], rows0_v, sem0).wait()
            pltpu.sync_copy(rows0_v, acc_sh.at[dst0_v], add=True)

            _load_idx(r, 2 * p + 2, dst0_v, srca0_v)
            pltpu.async_copy(xcat.at[srca0_v], rows0_v, sem0)
            pltpu.make_async_copy(xcat.at[srca1_v], rows1_v, sem1).wait()
            pltpu.sync_copy(rows1_v, acc_sh.at[dst1_v], add=True)
            return ()

        lax.fori_loop(0, (ITERS - 1) // 2, _edge_pair, ())
        pltpu.make_async_copy(xcat.at[srca0_v], rows0_v, sem0).wait()
        pltpu.sync_copy(rows0_v, acc_sh.at[dst0_v], add=True)
        plsc.subcore_barrier()

        pltpu.sync_copy(acc_sh.at[pl.ds(r0, RPT)],
                        a_hbm.at[c, pl.ds(r0, RPT)])
        plsc.subcore_barrier()

    pltpu.sync_copy(zacc.at[pl.ds(r0, RPT)], acc_sh.at[pl.ds(r0, RPT)])
    plsc.subcore_barrier()

    def _deg_iter(it, _):
        off = c * 2 * E + E + ebase + it * B
        pltpu.sync_copy(eboth.at[pl.ds(off, B)], dst0_v)
        pltpu.sync_copy(ones_v, acc_sh.at[dst0_v], add=True)
        return ()

    lax.fori_loop(0, ITERS, _deg_iter, ())
    plsc.subcore_barrier()
    pltpu.sync_copy(acc_sh.at[pl.ds(r0, RPT)], dg.at[c, pl.ds(r0, RPT)])


def _sc_aggregate(xcat, eboth):
    zacc = jnp.zeros((N_PAD, H), jnp.float32)
    ones = jnp.ones((B, H), jnp.float32)
    mesh = plsc.VectorSubcoreMesh(core_axis_name="c", subcore_axis_name="s")
    f = pl.kernel(
        _sc_body,
        out_type=(
            jax.ShapeDtypeStruct((2, N_PAD, H), jnp.float32),
            jax.ShapeDtypeStruct((2, N_PAD, H), jnp.float32),
            jax.ShapeDtypeStruct((2, N_PAD, H), jnp.float32),
        ),
        mesh=mesh,
        scratch_types=[
            pltpu.VMEM_SHARED((N_PAD, H), jnp.float32),
            pltpu.VMEM((B,), jnp.int32),
            pltpu.VMEM((B,), jnp.int32),
            pltpu.VMEM((B, H), jnp.float32),
            pltpu.VMEM((B,), jnp.int32),
            pltpu.VMEM((B,), jnp.int32),
            pltpu.VMEM((B, H), jnp.float32),
            pltpu.VMEM((B, H), jnp.float32),
            pltpu.SemaphoreType.DMA,
            pltpu.SemaphoreType.DMA,
        ],
    )
    return f(xcat, eboth, zacc, ones)


def _tc_body(a0, a1, d0, d1, w0, w1, bb0, bb1, o):
    n0 = 1.0 / jnp.maximum(d0[...], 1.0)
    n1 = 1.0 / jnp.maximum(d1[...], 1.0)
    acc = jnp.dot(a0[...] * n0, w0[...], preferred_element_type=jnp.float32)
    acc += jnp.dot(a1[...] * n1, w1[...], preferred_element_type=jnp.float32)
    o[...] = acc + bb0[...] + bb1[...]


def _tc_epilogue(agg0, agg1, deg0, deg1, W0, b0, W1, b1):
    R = 1000
    grid = (N // R,)
    return pl.pallas_call(
        _tc_body,
        grid=grid,
        in_specs=[
            pl.BlockSpec((R, D), lambda i: (i, 0)),
            pl.BlockSpec((R, D), lambda i: (i, 0)),
            pl.BlockSpec((R, 1), lambda i: (i, 0)),
            pl.BlockSpec((R, 1), lambda i: (i, 0)),
            pl.BlockSpec((D, D), lambda i: (0, 0)),
            pl.BlockSpec((D, D), lambda i: (0, 0)),
            pl.BlockSpec((1, D), lambda i: (0, 0)),
            pl.BlockSpec((1, D), lambda i: (0, 0)),
        ],
        out_specs=pl.BlockSpec((R, D), lambda i: (i, 0)),
        out_shape=jax.ShapeDtypeStruct((N, D), jnp.float32),
    )(agg0, agg1, deg0, deg1, W0, W1,
      b0.reshape(1, D), b1.reshape(1, D))


@jax.jit
def kernel(x, edge_index_rel0, edge_index_rel1, W0, b0, W1, b1):
    xcat = x.reshape(2 * N, H)
    eboth = jnp.concatenate([edge_index_rel0.reshape(2 * E),
                             edge_index_rel1.reshape(2 * E)])
    agg0, agg1, dg = _sc_aggregate(xcat, eboth)
    a0 = jnp.concatenate([agg0[0, :N], agg0[1, :N]], axis=1)
    a1 = jnp.concatenate([agg1[0, :N], agg1[1, :N]], axis=1)
    deg0 = dg[0, :N, 0:1]
    deg1 = dg[1, :N, 0:1]
    return _tc_epilogue(a0, a1, deg0, deg1, W0, b0, W1, b1)


# speedup vs baseline: 4.4339x; 1.0980x over previous
import jax
import jax.numpy as jnp
from jax import lax
from jax.experimental import pallas as pl
from jax.experimental.pallas import tpu as pltpu
from jax.experimental.pallas import tpu_sc as plsc

N = 10000
N_PAD = 10240
D = 256
H = 128
E = 160000
NS = 16
B = 80
EPT = E // NS
ITERS = EPT // B
RPT = N_PAD // NS


def _sc_body(xcat, eboth, zacc, ones,
             agg0, agg1, dg,
             acc_sh, dst0_v, srca0_v, rows0_v, dst1_v, srca1_v,
             rows1_v, ones_v, sem0, sem1):
    c = lax.axis_index("c")
    s = lax.axis_index("s")
    r0 = s * RPT
    ebase = s * EPT

    pltpu.sync_copy(ones, ones_v)

    def _load_idx(r, it, dst_v, srca_v):
        off = ebase + it * B
        pltpu.sync_copy(eboth.at[pl.ds(r * 2 * E + E + off, B)], dst_v)
        pltpu.sync_copy(eboth.at[pl.ds(r * 2 * E + off, B)], srca_v)
        for j in range(B // 16):
            sl = pl.ds(j * 16, 16)
            srca_v[sl] = srca_v[sl] * 2 + c

    for r, a_hbm in ((0, agg0), (1, agg1)):
        pltpu.sync_copy(zacc.at[pl.ds(r0, RPT)], acc_sh.at[pl.ds(r0, RPT)])
        plsc.subcore_barrier()

        _load_idx(r, 0, dst0_v, srca0_v)
        pltpu.async_copy(xcat.at[srca0_v], rows0_v, sem0)

        def _edge_pair(p, _, r=r):
            _load_idx(r, 2 * p + 1, dst1_v, srca1_v)
            pltpu.async_copy(xcat.at[srca1_v], rows1_v, sem1)
            pltpu.make_async_copy(xcat.at[srca0_v], rows0_v, sem0).wait()
            pltpu.sync_copy(rows0_v, acc_sh.at[dst0_v], add=True)

            _load_idx(r, 2 * p + 2, dst0_v, srca0_v)
            pltpu.async_copy(xcat.at[srca0_v], rows0_v, sem0)
            pltpu.make_async_copy(xcat.at[srca1_v], rows1_v, sem1).wait()
            pltpu.sync_copy(rows1_v, acc_sh.at[dst1_v], add=True)
            return ()

        lax.fori_loop(0, (ITERS - 1) // 2, _edge_pair, ())
        pltpu.make_async_copy(xcat.at[srca0_v], rows0_v, sem0).wait()
        pltpu.sync_copy(rows0_v, acc_sh.at[dst0_v], add=True)
        plsc.subcore_barrier()

        pltpu.sync_copy(acc_sh.at[pl.ds(r0, RPT)],
                        a_hbm.at[c, pl.ds(r0, RPT)])
        plsc.subcore_barrier()

    pltpu.sync_copy(zacc.at[pl.ds(r0, RPT)], acc_sh.at[pl.ds(r0, RPT)])
    plsc.subcore_barrier()

    dbase = c * 2 * E + E + ebase

    pltpu.async_copy(eboth.at[pl.ds(dbase, B)], dst0_v, sem0)

    def _deg_pair(p, _):
        pltpu.async_copy(eboth.at[pl.ds(dbase + (2 * p + 1) * B, B)],
                         dst1_v, sem1)
        pltpu.make_async_copy(eboth.at[pl.ds(dbase + 2 * p * B, B)],
                              dst0_v, sem0).wait()
        pltpu.sync_copy(ones_v, acc_sh.at[dst0_v], add=True)

        pltpu.async_copy(eboth.at[pl.ds(dbase + (2 * p + 2) * B, B)],
                         dst0_v, sem0)
        pltpu.make_async_copy(eboth.at[pl.ds(dbase + (2 * p + 1) * B, B)],
                              dst1_v, sem1).wait()
        pltpu.sync_copy(ones_v, acc_sh.at[dst1_v], add=True)
        return ()

    lax.fori_loop(0, (ITERS - 1) // 2, _deg_pair, ())
    pltpu.make_async_copy(eboth.at[pl.ds(dbase + (ITERS - 1) * B, B)],
                          dst0_v, sem0).wait()
    pltpu.sync_copy(ones_v, acc_sh.at[dst0_v], add=True)
    plsc.subcore_barrier()
    pltpu.sync_copy(acc_sh.at[pl.ds(r0, RPT)], dg.at[c, pl.ds(r0, RPT)])


def _sc_aggregate(xcat, eboth):
    zacc = jnp.zeros((N_PAD, H), jnp.float32)
    ones = jnp.ones((B, H), jnp.float32)
    mesh = plsc.VectorSubcoreMesh(core_axis_name="c", subcore_axis_name="s")
    f = pl.kernel(
        _sc_body,
        out_type=(
            jax.ShapeDtypeStruct((2, N_PAD, H), jnp.float32),
            jax.ShapeDtypeStruct((2, N_PAD, H), jnp.float32),
            jax.ShapeDtypeStruct((2, N_PAD, H), jnp.float32),
        ),
        mesh=mesh,
        scratch_types=[
            pltpu.VMEM_SHARED((N_PAD, H), jnp.float32),
            pltpu.VMEM((B,), jnp.int32),
            pltpu.VMEM((B,), jnp.int32),
            pltpu.VMEM((B, H), jnp.float32),
            pltpu.VMEM((B,), jnp.int32),
            pltpu.VMEM((B,), jnp.int32),
            pltpu.VMEM((B, H), jnp.float32),
            pltpu.VMEM((B, H), jnp.float32),
            pltpu.SemaphoreType.DMA,
            pltpu.SemaphoreType.DMA,
        ],
    )
    return f(xcat, eboth, zacc, ones)


def _tc_body(a0, a1, d0, d1, w0, w1, bb0, bb1, o):
    n0 = 1.0 / jnp.maximum(d0[...], 1.0)
    n1 = 1.0 / jnp.maximum(d1[...], 1.0)
    acc = jnp.dot(a0[...] * n0, w0[...], preferred_element_type=jnp.float32)
    acc += jnp.dot(a1[...] * n1, w1[...], preferred_element_type=jnp.float32)
    o[...] = acc + bb0[...] + bb1[...]


def _tc_epilogue(agg0, agg1, deg0, deg1, W0, b0, W1, b1):
    R = 1000
    grid = (N // R,)
    return pl.pallas_call(
        _tc_body,
        grid=grid,
        in_specs=[
            pl.BlockSpec((R, D), lambda i: (i, 0)),
            pl.BlockSpec((R, D), lambda i: (i, 0)),
            pl.BlockSpec((R, 1), lambda i: (i, 0)),
            pl.BlockSpec((R, 1), lambda i: (i, 0)),
            pl.BlockSpec((D, D), lambda i: (0, 0)),
            pl.BlockSpec((D, D), lambda i: (0, 0)),
            pl.BlockSpec((1, D), lambda i: (0, 0)),
            pl.BlockSpec((1, D), lambda i: (0, 0)),
        ],
        out_specs=pl.BlockSpec((R, D), lambda i: (i, 0)),
        out_shape=jax.ShapeDtypeStruct((N, D), jnp.float32),
    )(agg0, agg1, deg0, deg1, W0, W1,
      b0.reshape(1, D), b1.reshape(1, D))


@jax.jit
def kernel(x, edge_index_rel0, edge_index_rel1, W0, b0, W1, b1):
    xcat = x.reshape(2 * N, H)
    eboth = jnp.concatenate([edge_index_rel0.reshape(2 * E),
                             edge_index_rel1.reshape(2 * E)])
    agg0, agg1, dg = _sc_aggregate(xcat, eboth)
    a0 = jnp.concatenate([agg0[0, :N], agg0[1, :N]], axis=1)
    a1 = jnp.concatenate([agg1[0, :N], agg1[1, :N]], axis=1)
    deg0 = dg[0, :N, 0:1]
    deg1 = dg[1, :N, 0:1]
    return _tc_epilogue(a0, a1, deg0, deg1, W0, b0, W1, b1)
